# Initial kernel scaffold; baseline (speedup 1.0000x reference)
import jax
import jax.numpy as jnp
from jax.experimental import pallas as pl


def kernel(input):
    raise NotImplementedError("write your pallas kernel here")


# speedup vs baseline: 1.3203x; 1.3203x over previous
import functools

import jax
import jax.numpy as jnp
from jax import lax
from jax.experimental import pallas as pl
from jax.experimental.pallas import tpu as pltpu
from jax.experimental.pallas import tpu_sc as plsc

_R = 3
_BW = 0.1
_C, _H, _W = 96, 224, 224
_HW = _H * _W
_PAD = 16
_LANES = 16
_NWORK = 32
_CPW = _C // _NWORK
_NVEC = _W // _LANES


def _body(x_hbm, out_hbm, xbuf, obuf, sem):
    del sem
    wid = lax.axis_index("s") * 2 + lax.axis_index("c")
    zero16 = jnp.zeros((_LANES,), jnp.float32)
    lane = lax.iota(jnp.int32, _LANES)

    for k in range(_CPW):
        ch = wid * _CPW + k
        pltpu.sync_copy(x_hbm.at[ch], xbuf.at[pl.ds(_PAD, _HW)])

        for jv in range(_NVEC):
            obuf[pl.ds(jv * _LANES, _LANES)] = zero16
            obuf[pl.ds((_H - 1) * _W + jv * _LANES, _LANES)] = zero16

        for jv in range(_NVEC):
            col0 = jv * _LANES

            def row_body(i, _, col0=col0, jv=jv):
                base = i * _W + col0 + _PAD
                c = xbuf[pl.ds(base, _LANES)]
                acc = jnp.full((_LANES,), 1.0, jnp.float32)
                for di in (-1, 0, 1):
                    for dj in (-1, 0, 1):
                        if di == 0 and dj == 0:
                            continue
                        v = xbuf[pl.ds(base + di * _W + dj, _LANES)]
                        d = jnp.abs(v - c)
                        acc = acc + jnp.maximum(0.0, 1.0 - d * (1.0 / _BW))
                acc = acc * jnp.float32(1.0 / (_R * _R))
                if jv == 0:
                    acc = jnp.where(lane >= 1, acc, 0.0)
                if jv == _NVEC - 1:
                    acc = jnp.where(lane <= _LANES - 2, acc, 0.0)
                obuf[pl.ds(i * _W + col0, _LANES)] = acc
                return 0

            lax.fori_loop(1, _H - 1, row_body, 0)

        pltpu.sync_copy(obuf, out_hbm.at[ch])


@jax.jit
def _hist_sc(x2d):
    mesh = plsc.VectorSubcoreMesh(core_axis_name="c", subcore_axis_name="s")
    f = pl.kernel(
        _body,
        out_type=jax.ShapeDtypeStruct((_C, _HW), jnp.float32),
        mesh=mesh,
        scratch_types=[
            pltpu.VMEM((_PAD + _HW + _PAD,), jnp.float32),
            pltpu.VMEM((_HW,), jnp.float32),
            pltpu.SemaphoreType.DMA,
        ],
        compiler_params=pltpu.CompilerParams(use_tc_tiling_on_sc=False),
    )
    return f(x2d)


def kernel(input):
    n, sf, c, h, w = input.shape
    x2d = input.reshape(_C, _HW)
    out = _hist_sc(x2d)
    return out.reshape(n, sf, c, h, w)


# speedup vs baseline: 1.4437x; 1.0934x over previous
import functools

import jax
import jax.numpy as jnp
from jax import lax
from jax.experimental import pallas as pl
from jax.experimental.pallas import tpu as pltpu
from jax.experimental.pallas import tpu_sc as plsc

_R = 3
_BW = 0.1
_C, _H, _W = 96, 224, 224
_HW = _H * _W
_PAD = 16
_LANES = 16
_NWORK = 32
_CPW = _C // _NWORK
_NVEC = _W // _LANES


def _body(x_hbm, out_hbm, xbuf, obuf, sem):
    del sem
    wid = lax.axis_index("s") * 2 + lax.axis_index("c")
    zero16 = jnp.zeros((_LANES,), jnp.float32)
    lane = lax.iota(jnp.int32, _LANES)

    for k in range(_CPW):
        ch = wid * _CPW + k
        pltpu.sync_copy(x_hbm.at[ch], xbuf.at[pl.ds(_PAD, _HW)])

        for jv in range(_NVEC):
            obuf[pl.ds(jv * _LANES, _LANES)] = zero16
            obuf[pl.ds((_H - 1) * _W + jv * _LANES, _LANES)] = zero16

        for jv in range(_NVEC):
            col0 = jv * _LANES

            def row_body(i, _, col0=col0, jv=jv):
                base = i * _W + col0 + _PAD
                c = xbuf[pl.ds(base, _LANES)]
                acc = jnp.zeros((_LANES,), jnp.float32)
                for di in (-1, 0, 1):
                    for dj in (-1, 0, 1):
                        if di == 0 and dj == 0:
                            continue
                        v = xbuf[pl.ds(base + di * _W + dj, _LANES)]
                        acc = acc + jnp.maximum(0.0, _BW - jnp.abs(v - c))
                acc = acc * jnp.float32(1.0 / (_BW * _R * _R)) + jnp.float32(
                    1.0 / (_R * _R))
                if jv == 0:
                    acc = jnp.where(lane >= 1, acc, 0.0)
                if jv == _NVEC - 1:
                    acc = jnp.where(lane <= _LANES - 2, acc, 0.0)
                obuf[pl.ds(i * _W + col0, _LANES)] = acc
                return 0

            lax.fori_loop(1, _H - 1, row_body, 0)

        pltpu.sync_copy(obuf, out_hbm.at[ch])


@jax.jit
def _hist_sc(x2d):
    mesh = plsc.VectorSubcoreMesh(core_axis_name="c", subcore_axis_name="s")
    f = pl.kernel(
        _body,
        out_type=jax.ShapeDtypeStruct((_C, _HW), jnp.float32),
        mesh=mesh,
        scratch_types=[
            pltpu.VMEM((_PAD + _HW + _PAD,), jnp.float32),
            pltpu.VMEM((_HW,), jnp.float32),
            pltpu.SemaphoreType.DMA,
        ],
        compiler_params=pltpu.CompilerParams(use_tc_tiling_on_sc=False),
    )
    return f(x2d)


def kernel(input):
    n, sf, c, h, w = input.shape
    x2d = input.reshape(_C, _HW)
    out = _hist_sc(x2d)
    return out.reshape(n, sf, c, h, w)
